# baseline (speedup 1.0000x reference)
import functools

import jax
import jax.numpy as jnp
from jax import lax
from jax.experimental import pallas as pl
from jax.experimental.pallas import tpu as pltpu
from jax.experimental.pallas import tpu_sc as plsc

VOCAB = 30000
EMB = 128
D = 768
E = 8
K = 3
DFF = 768
T = 2048
CAP = 1024

TB1 = 256
NB = 3072
NEG = -1e30


_NW = 32
_BPW = T // _NW


def _sc_gather(table, idx):
    mesh = plsc.VectorSubcoreMesh(core_axis_name="c", subcore_axis_name="s")

    @functools.partial(
        pl.kernel,
        mesh=mesh,
        out_type=jax.ShapeDtypeStruct((T, EMB), jnp.float32),
        scratch_types=[
            pltpu.VMEM((_BPW,), jnp.int32),
            pltpu.VMEM((_BPW, EMB), jnp.float32),
            pltpu.SemaphoreType.DMA,
        ],
    )
    def k(table_hbm, idx_hbm, out_hbm, idx_v, rows_v, sem):
        wid = lax.axis_index("s") * 2 + lax.axis_index("c")
        base = wid * _BPW
        pltpu.sync_copy(idx_hbm.at[pl.ds(base, _BPW)], idx_v)
        pltpu.async_copy(table_hbm.at[idx_v], rows_v, sem).wait()
        pltpu.sync_copy(rows_v, out_hbm.at[pl.ds(base, _BPW)])

    return k(table, idx)


def _core_body(rows_ref, pos_ref, type_ref, g_ref, b_ref, win_ref, bin_ref,
               wg_ref, w1_ref, b1_ref, w2_ref, b2_ref, lmw_ref, lmb_ref,
               lmg_ref, lmbb_ref, xm_ref, hid_ref, xs_ref, ws_ref, acc_ref):
    e = pl.program_id(0)

    @pl.when(e == 0)
    def _():
        emb = rows_ref[...] + pos_ref[...] + type_ref[...]
        mu = jnp.mean(emb, axis=1, keepdims=True)
        d = emb - mu
        var = jnp.mean(d * d, axis=1, keepdims=True)
        ln = d * lax.rsqrt(var + 1e-12) * g_ref[...] + b_ref[...]

        x = jnp.dot(ln, win_ref[...], preferred_element_type=jnp.float32)
        x = x + bin_ref[...]
        xs_ref[...] = x

        logits = jnp.dot(x, wg_ref[...], preferred_element_type=jnp.float32)
        lane = lax.broadcasted_iota(jnp.int32, (T, 128), 1)
        logits = jnp.where(lane < E, logits, NEG)

        cur = logits
        vals, ohs = [], []
        for _ in range(K):
            vk = jnp.max(cur, axis=1, keepdims=True)
            ik = jnp.min(jnp.where(cur == vk, lane, 128), axis=1, keepdims=True)
            oh = (lane == ik).astype(jnp.float32)
            vals.append(vk)
            ohs.append(oh)
            cur = jnp.where(lane == ik, NEG, cur)

        es = [jnp.exp(v - vals[0]) for v in vals]
        denom = es[0] + es[1] + es[2]
        gates = [ex / denom for ex in es]

        c = (ohs[0] + ohs[1] + ohs[2]).astype(jnp.bfloat16)
        ri = lax.broadcasted_iota(jnp.int32, (T, T), 0)
        rj = lax.broadcasted_iota(jnp.int32, (T, T), 1)
        lt = (rj < ri).astype(jnp.bfloat16)
        pos = jnp.dot(lt, c, preferred_element_type=jnp.float32)

        w = jnp.zeros((T, 128), jnp.float32)
        for kk in range(K):
            pk = jnp.sum(ohs[kk] * pos, axis=1, keepdims=True)
            valid = (pk < CAP).astype(jnp.float32)
            w = w + gates[kk] * valid * ohs[kk]
        ws_ref[...] = w

    x = xs_ref[...]
    h = jnp.dot(x, w1_ref[0], preferred_element_type=jnp.float32) + b1_ref[0]
    h = jax.nn.gelu(h)
    y = jnp.dot(h, w2_ref[0], preferred_element_type=jnp.float32) + b2_ref[0]
    lane2 = lax.broadcasted_iota(jnp.int32, (T, 128), 1)
    we = jnp.sum(jnp.where(lane2 == e, ws_ref[...], 0.0), axis=1, keepdims=True)
    contrib = we * y

    @pl.when(e == 0)
    def _():
        acc_ref[...] = contrib

    @pl.when(e > 0)
    def _():
        acc_ref[...] = acc_ref[...] + contrib

    @pl.when(e == E - 1)
    def _():
        xm = acc_ref[...]
        xm_ref[...] = xm
        hh = jnp.dot(xm, lmw_ref[...], preferred_element_type=jnp.float32)
        hh = jax.nn.gelu(hh + lmb_ref[...])
        mu = jnp.mean(hh, axis=1, keepdims=True)
        d = hh - mu
        var = jnp.mean(d * d, axis=1, keepdims=True)
        hid_ref[...] = d * lax.rsqrt(var + 1e-12) * lmg_ref[...] + lmbb_ref[...]


def _core(rows, pos_emb, type_row, ln_g, ln_b, W_in, b_in, W_gate_p,
          W1, b1, W2, b2, lm_dense, lm_dense_b, lm_ln_g, lm_ln_b):
    full2 = lambda e: (0, 0)
    return pl.pallas_call(
        _core_body,
        grid=(E,),
        in_specs=[
            pl.BlockSpec((T, EMB), full2),
            pl.BlockSpec((T, EMB), full2),
            pl.BlockSpec((1, EMB), full2),
            pl.BlockSpec((1, EMB), full2),
            pl.BlockSpec((1, EMB), full2),
            pl.BlockSpec((EMB, D), full2),
            pl.BlockSpec((1, D), full2),
            pl.BlockSpec((D, 128), full2),
            pl.BlockSpec((1, D, DFF), lambda e: (e, 0, 0)),
            pl.BlockSpec((1, 1, DFF), lambda e: (e, 0, 0)),
            pl.BlockSpec((1, DFF, D), lambda e: (e, 0, 0)),
            pl.BlockSpec((1, 1, D), lambda e: (e, 0, 0)),
            pl.BlockSpec((D, EMB), full2),
            pl.BlockSpec((1, EMB), full2),
            pl.BlockSpec((1, EMB), full2),
            pl.BlockSpec((1, EMB), full2),
        ],
        out_specs=[
            pl.BlockSpec((T, D), full2),
            pl.BlockSpec((T, EMB), full2),
        ],
        out_shape=[
            jax.ShapeDtypeStruct((T, D), jnp.float32),
            jax.ShapeDtypeStruct((T, EMB), jnp.float32),
        ],
        scratch_shapes=[
            pltpu.VMEM((T, D), jnp.float32),
            pltpu.VMEM((T, 128), jnp.float32),
            pltpu.VMEM((T, D), jnp.float32),
        ],
    )(rows, pos_emb, type_row, ln_g, ln_b, W_in, b_in, W_gate_p,
      W1, b1.reshape(E, 1, DFF), W2, b2.reshape(E, 1, D),
      lm_dense, lm_dense_b.reshape(1, EMB), lm_ln_g.reshape(1, EMB),
      lm_ln_b.reshape(1, EMB))


def _dec_body(hid_ref, dec_ref, db_ref, out_ref):
    out_ref[...] = (
        jnp.dot(hid_ref[...], dec_ref[...], preferred_element_type=jnp.float32)
        + db_ref[...]
    )


def _dec(hid, decoder, decoder_b):
    nblk = pl.cdiv(VOCAB, NB)
    return pl.pallas_call(
        _dec_body,
        grid=(nblk,),
        in_specs=[
            pl.BlockSpec((T, EMB), lambda j: (0, 0)),
            pl.BlockSpec((EMB, NB), lambda j: (0, j)),
            pl.BlockSpec((1, NB), lambda j: (0, j)),
        ],
        out_specs=pl.BlockSpec((T, NB), lambda j: (0, j)),
        out_shape=jax.ShapeDtypeStruct((T, VOCAB), jnp.float32),
    )(hid, decoder, decoder_b.reshape(1, VOCAB))


def kernel(input_ids, word_emb, pos_emb, type_emb, ln_emb_g, ln_emb_b, W_in,
           b_in, W_gate, W1, b1, W2, b2, lm_dense, lm_dense_b, lm_ln_g,
           lm_ln_b, decoder, decoder_b):
    idx = input_ids.reshape(T).astype(jnp.int32)
    rows = _sc_gather(word_emb, idx)

    W_gate_p = jnp.pad(W_gate, ((0, 0), (0, 128 - E)))
    xm, hid = _core(rows, pos_emb, type_emb[0:1, :], ln_emb_g.reshape(1, EMB),
                    ln_emb_b.reshape(1, EMB), W_in, b_in.reshape(1, D),
                    W_gate_p, W1, b1, W2, b2, lm_dense, lm_dense_b,
                    lm_ln_g, lm_ln_b)
    scores = _dec(hid, decoder, decoder_b)
    return scores.reshape(1, T, VOCAB), xm.reshape(1, T, D)


# speedup vs baseline: 1.0007x; 1.0007x over previous
import functools

import jax
import jax.numpy as jnp
from jax import lax
from jax.experimental import pallas as pl
from jax.experimental.pallas import tpu as pltpu
from jax.experimental.pallas import tpu_sc as plsc

VOCAB = 30000
EMB = 128
D = 768
E = 8
K = 3
DFF = 768
T = 2048
CAP = 1024

NB = 3072
NEG = -1e30


_NW = 32
_BPW = T // _NW


def _sc_gather(table, idx):
    mesh = plsc.VectorSubcoreMesh(core_axis_name="c", subcore_axis_name="s")

    @functools.partial(
        pl.kernel,
        mesh=mesh,
        out_type=jax.ShapeDtypeStruct((T, EMB), jnp.float32),
        scratch_types=[
            pltpu.VMEM((_BPW,), jnp.int32),
            pltpu.VMEM((_BPW, EMB), jnp.float32),
            pltpu.SemaphoreType.DMA,
        ],
    )
    def k(table_hbm, idx_hbm, out_hbm, idx_v, rows_v, sem):
        wid = lax.axis_index("s") * 2 + lax.axis_index("c")
        base = wid * _BPW
        pltpu.sync_copy(idx_hbm.at[pl.ds(base, _BPW)], idx_v)
        pltpu.async_copy(table_hbm.at[idx_v], rows_v, sem).wait()
        pltpu.sync_copy(rows_v, out_hbm.at[pl.ds(base, _BPW)])

    return k(table, idx)


def _core_body(rows_ref, pos_ref, type_ref, g_ref, b_ref, win_ref, bin_ref,
               wg_ref, w1_ref, b1_ref, w2_ref, b2_ref, lmw_ref, lmb_ref,
               lmg_ref, lmbb_ref, xm_ref, hid_ref, xs_ref, ws_ref, acc_ref):
    e = pl.program_id(0)

    @pl.when(e == 0)
    def _():
        emb = rows_ref[...] + pos_ref[...] + type_ref[...]
        mu = jnp.mean(emb, axis=1, keepdims=True)
        d = emb - mu
        var = jnp.mean(d * d, axis=1, keepdims=True)
        ln = d * lax.rsqrt(var + 1e-12) * g_ref[...] + b_ref[...]

        x = jnp.dot(ln, win_ref[...], preferred_element_type=jnp.float32)
        x = x + bin_ref[...]
        xs_ref[...] = x

        logits = jnp.dot(x, wg_ref[...], preferred_element_type=jnp.float32)
        lane = lax.broadcasted_iota(jnp.int32, (T, 128), 1)
        logits = jnp.where(lane < E, logits, NEG)

        cur = logits
        vals, ohs = [], []
        for _ in range(K):
            vk = jnp.max(cur, axis=1, keepdims=True)
            ik = jnp.min(jnp.where(cur == vk, lane, 128), axis=1, keepdims=True)
            oh = (lane == ik).astype(jnp.float32)
            vals.append(vk)
            ohs.append(oh)
            cur = jnp.where(lane == ik, NEG, cur)

        es = [jnp.exp(v - vals[0]) for v in vals]
        denom = es[0] + es[1] + es[2]
        gates = [ex / denom for ex in es]

        c = (ohs[0] + ohs[1] + ohs[2]).astype(jnp.bfloat16)
        ri = lax.broadcasted_iota(jnp.int32, (T, T), 0)
        rj = lax.broadcasted_iota(jnp.int32, (T, T), 1)
        lt = (rj < ri).astype(jnp.bfloat16)
        pos = jnp.dot(lt, c, preferred_element_type=jnp.float32)

        w = jnp.zeros((T, 128), jnp.float32)
        for kk in range(K):
            pk = jnp.sum(ohs[kk] * pos, axis=1, keepdims=True)
            valid = (pk < CAP).astype(jnp.float32)
            w = w + gates[kk] * valid * ohs[kk]
        ws_ref[...] = w

    x = xs_ref[...]
    h = jnp.dot(x, w1_ref[0], preferred_element_type=jnp.float32) + b1_ref[0]
    h = jax.nn.gelu(h)
    y = jnp.dot(h, w2_ref[0], preferred_element_type=jnp.float32) + b2_ref[0]
    lane2 = lax.broadcasted_iota(jnp.int32, (T, 128), 1)
    we = jnp.sum(jnp.where(lane2 == e, ws_ref[...], 0.0), axis=1, keepdims=True)
    contrib = we * y

    @pl.when(e == 0)
    def _():
        acc_ref[...] = contrib

    @pl.when(e > 0)
    def _():
        acc_ref[...] = acc_ref[...] + contrib

    @pl.when(e == E - 1)
    def _():
        xm = acc_ref[...]
        xm_ref[...] = xm
        hh = jnp.dot(xm, lmw_ref[...], preferred_element_type=jnp.float32)
        hh = jax.nn.gelu(hh + lmb_ref[...])
        mu = jnp.mean(hh, axis=1, keepdims=True)
        d = hh - mu
        var = jnp.mean(d * d, axis=1, keepdims=True)
        hid_ref[...] = d * lax.rsqrt(var + 1e-12) * lmg_ref[...] + lmbb_ref[...]


def _core(rows, pos_emb, type_row, ln_g, ln_b, W_in, b_in, W_gate_p,
          W1, b1, W2, b2, lm_dense, lm_dense_b, lm_ln_g, lm_ln_b):
    full2 = lambda e: (0, 0)
    return pl.pallas_call(
        _core_body,
        grid=(E,),
        in_specs=[
            pl.BlockSpec((T, EMB), full2),
            pl.BlockSpec((T, EMB), full2),
            pl.BlockSpec((1, EMB), full2),
            pl.BlockSpec((1, EMB), full2),
            pl.BlockSpec((1, EMB), full2),
            pl.BlockSpec((EMB, D), full2),
            pl.BlockSpec((1, D), full2),
            pl.BlockSpec((D, 128), full2),
            pl.BlockSpec((1, D, DFF), lambda e: (e, 0, 0)),
            pl.BlockSpec((1, 1, DFF), lambda e: (e, 0, 0)),
            pl.BlockSpec((1, DFF, D), lambda e: (e, 0, 0)),
            pl.BlockSpec((1, 1, D), lambda e: (e, 0, 0)),
            pl.BlockSpec((D, EMB), full2),
            pl.BlockSpec((1, EMB), full2),
            pl.BlockSpec((1, EMB), full2),
            pl.BlockSpec((1, EMB), full2),
        ],
        out_specs=[
            pl.BlockSpec((T, D), full2),
            pl.BlockSpec((T, EMB), full2),
        ],
        out_shape=[
            jax.ShapeDtypeStruct((T, D), jnp.float32),
            jax.ShapeDtypeStruct((T, EMB), jnp.float32),
        ],
        scratch_shapes=[
            pltpu.VMEM((T, D), jnp.float32),
            pltpu.VMEM((T, 128), jnp.float32),
            pltpu.VMEM((T, D), jnp.float32),
        ],
    )(rows, pos_emb, type_row, ln_g, ln_b, W_in, b_in, W_gate_p,
      W1, b1.reshape(E, 1, DFF), W2, b2.reshape(E, 1, D),
      lm_dense, lm_dense_b.reshape(1, EMB), lm_ln_g.reshape(1, EMB),
      lm_ln_b.reshape(1, EMB))


def _dec_body(hid_ref, dec_ref, db_ref, out_ref):
    out_ref[...] = (
        jnp.dot(hid_ref[...], dec_ref[...], preferred_element_type=jnp.float32)
        + db_ref[...]
    )


def _dec(hid, decoder, decoder_b):
    nblk = pl.cdiv(VOCAB, NB)
    return pl.pallas_call(
        _dec_body,
        grid=(nblk,),
        in_specs=[
            pl.BlockSpec((T, EMB), lambda j: (0, 0)),
            pl.BlockSpec((EMB, NB), lambda j: (0, j)),
            pl.BlockSpec((1, NB), lambda j: (0, j)),
        ],
        out_specs=pl.BlockSpec((T, NB), lambda j: (0, j)),
        out_shape=jax.ShapeDtypeStruct((T, VOCAB), jnp.float32),
    )(hid, decoder, decoder_b.reshape(1, VOCAB))


def kernel(input_ids, word_emb, pos_emb, type_emb, ln_emb_g, ln_emb_b, W_in,
           b_in, W_gate, W1, b1, W2, b2, lm_dense, lm_dense_b, lm_ln_g,
           lm_ln_b, decoder, decoder_b):
    idx = input_ids.reshape(T).astype(jnp.int32)
    rows = _sc_gather(word_emb, idx)

    W_gate_p = jnp.pad(W_gate, ((0, 0), (0, 128 - E)))
    xm, hid = _core(rows, pos_emb, type_emb[0:1, :], ln_emb_g.reshape(1, EMB),
                    ln_emb_b.reshape(1, EMB), W_in, b_in.reshape(1, D),
                    W_gate_p, W1, b1, W2, b2, lm_dense, lm_dense_b,
                    lm_ln_g, lm_ln_b)
    scores = _dec(hid, decoder, decoder_b)
    return scores.reshape(1, T, VOCAB), xm.reshape(1, T, D)
